# baseline (speedup 1.0000x reference)
import jax
import jax.numpy as jnp
from jax import lax
from jax.experimental import pallas as pl
from jax.experimental.pallas import tpu as pltpu
from jax.experimental.pallas import tpu_sc as plsc

N_TOP = 100000
N_CLS = 128
B = 16384

NUM_CORES = 2
NUM_SUBCORES = 16
NUM_WORKERS = NUM_CORES * NUM_SUBCORES
LANES = 16
CHUNK = 128
ROWS_PER_W = B // NUM_WORKERS
N_CHUNKS = ROWS_PER_W // CHUNK


def _calib_body(logits_hbm, topics_hbm, loga_hbm, b_hbm, out_hbm,
                idx_v, scale_v, rows_v, logits_v, sem_rows, sem_scale):
    wid = lax.axis_index("s") * NUM_CORES + lax.axis_index("c")
    base = wid * ROWS_PER_W

    def chunk_body(c, _):
        off = pl.multiple_of(base + c * CHUNK, CHUNK)
        pltpu.sync_copy(topics_hbm.at[pl.ds(off, CHUNK)], idx_v)
        cp_rows = pltpu.async_copy(b_hbm.at[idx_v], rows_v, sem_rows)
        cp_scale = pltpu.async_copy(loga_hbm.at[idx_v], scale_v, sem_scale)
        pltpu.sync_copy(logits_hbm.at[pl.ds(off, CHUNK)], logits_v)
        cp_scale.wait()
        for j in range(CHUNK // LANES):
            sl = pl.ds(j * LANES, LANES)
            scale_v[sl] = jnp.exp(scale_v[sl])
        cp_rows.wait()

        def group_body(g, _):
            sv = scale_v[pl.ds(g * LANES, LANES)]
            for r in range(LANES):
                i = g * LANES + r
                s = sv[r]
                for k in range(N_CLS // LANES):
                    sl = pl.ds(k * LANES, LANES)
                    rows_v[i, sl] = logits_v[i, sl] * s + rows_v[i, sl]
            return 0

        lax.fori_loop(0, CHUNK // LANES, group_body, 0)
        pltpu.sync_copy(rows_v, out_hbm.at[pl.ds(off, CHUNK)])
        return 0

    lax.fori_loop(0, N_CHUNKS, chunk_body, 0)


@jax.jit
def kernel(logits, topics, loga, b):
    topics = topics.astype(jnp.int32)
    run = pl.kernel(
        _calib_body,
        out_type=jax.ShapeDtypeStruct((B, N_CLS), jnp.float32),
        mesh=plsc.VectorSubcoreMesh(core_axis_name="c", subcore_axis_name="s"),
        scratch_types=[
            pltpu.VMEM((CHUNK,), jnp.int32),
            pltpu.VMEM((CHUNK,), jnp.float32),
            pltpu.VMEM((CHUNK, N_CLS), jnp.float32),
            pltpu.VMEM((CHUNK, N_CLS), jnp.float32),
            pltpu.SemaphoreType.DMA,
            pltpu.SemaphoreType.DMA,
        ],
    )
    return run(logits, topics, loga, b)


# speedup vs baseline: 1.0390x; 1.0390x over previous
import jax
import jax.numpy as jnp
from jax import lax
from jax.experimental import pallas as pl
from jax.experimental.pallas import tpu as pltpu
from jax.experimental.pallas import tpu_sc as plsc

N_TOP = 100000
N_CLS = 128
B = 16384

NUM_CORES = 2
NUM_SUBCORES = 16
NUM_WORKERS = NUM_CORES * NUM_SUBCORES
LANES = 16
CHUNK = 128
ROWS_PER_W = B // NUM_WORKERS
N_CHUNKS = ROWS_PER_W // CHUNK


def _calib_body(logits_hbm, topics_hbm, loga_hbm, b_hbm, out_hbm,
                idx_v, scale_v,
                rows0, rows1, logits0, logits1, out0, out1,
                sem_r0, sem_r1, sem_s0, sem_s1,
                sem_l0, sem_l1, sem_o0, sem_o1):
    wid = lax.axis_index("s") * NUM_CORES + lax.axis_index("c")
    base = wid * ROWS_PER_W

    rows = (rows0, rows1)
    logits_b = (logits0, logits1)
    out_b = (out0, out1)
    sem_r = (sem_r0, sem_r1)
    sem_s = (sem_s0, sem_s1)
    sem_l = (sem_l0, sem_l1)
    sem_o = (sem_o0, sem_o1)

    pltpu.sync_copy(topics_hbm.at[pl.ds(pl.multiple_of(base, CHUNK), ROWS_PER_W)], idx_v)

    def issue(c):
        p = c % 2
        isl = pl.ds(c * CHUNK, CHUNK)
        off = pl.ds(pl.multiple_of(base + c * CHUNK, CHUNK), CHUNK)
        r = pltpu.async_copy(b_hbm.at[idx_v.at[isl]], rows[p], sem_r[p])
        s = pltpu.async_copy(loga_hbm.at[idx_v.at[isl]], scale_v.at[isl], sem_s[p])
        l = pltpu.async_copy(logits_hbm.at[off], logits_b[p], sem_l[p])
        return (r, s, l)

    in_cp = [None] * N_CHUNKS
    out_cp = [None] * N_CHUNKS
    in_cp[0] = issue(0)
    in_cp[1] = issue(1)

    for c in range(N_CHUNKS):
        p = c % 2
        if c >= 2:
            out_cp[c - 2].wait()
        r, s, l = in_cp[c]
        s.wait()
        for j in range(CHUNK // LANES):
            sl = pl.ds(c * CHUNK + j * LANES, LANES)
            scale_v[sl] = jnp.exp(scale_v[sl])
        r.wait()
        l.wait()

        def group_body(g, _, c=c, p=p):
            sv = scale_v[pl.ds(c * CHUNK + g * LANES, LANES)]
            for rr in range(LANES):
                i = g * LANES + rr
                sc = sv[rr]
                for k in range(N_CLS // LANES):
                    sl = pl.ds(k * LANES, LANES)
                    out_b[p][i, sl] = logits_b[p][i, sl] * sc + rows[p][i, sl]
            return 0

        lax.fori_loop(0, CHUNK // LANES, group_body, 0)
        off = pl.ds(pl.multiple_of(base + c * CHUNK, CHUNK), CHUNK)
        out_cp[c] = pltpu.async_copy(out_b[p], out_hbm.at[off], sem_o[p])
        if c + 2 < N_CHUNKS:
            in_cp[c + 2] = issue(c + 2)

    out_cp[N_CHUNKS - 2].wait()
    out_cp[N_CHUNKS - 1].wait()


@jax.jit
def kernel(logits, topics, loga, b):
    topics = topics.astype(jnp.int32)
    run = pl.kernel(
        _calib_body,
        out_type=jax.ShapeDtypeStruct((B, N_CLS), jnp.float32),
        mesh=plsc.VectorSubcoreMesh(core_axis_name="c", subcore_axis_name="s"),
        scratch_types=[
            pltpu.VMEM((ROWS_PER_W,), jnp.int32),
            pltpu.VMEM((ROWS_PER_W,), jnp.float32),
            pltpu.VMEM((CHUNK, N_CLS), jnp.float32),
            pltpu.VMEM((CHUNK, N_CLS), jnp.float32),
            pltpu.VMEM((CHUNK, N_CLS), jnp.float32),
            pltpu.VMEM((CHUNK, N_CLS), jnp.float32),
            pltpu.VMEM((CHUNK, N_CLS), jnp.float32),
            pltpu.VMEM((CHUNK, N_CLS), jnp.float32),
        ] + [pltpu.SemaphoreType.DMA] * 8,
    )
    return run(logits, topics, loga, b)


# speedup vs baseline: 1.2209x; 1.1750x over previous
import jax
import jax.numpy as jnp
from jax import lax
from jax.experimental import pallas as pl
from jax.experimental.pallas import tpu as pltpu
from jax.experimental.pallas import tpu_sc as plsc

N_TOP = 100000
N_CLS = 128
B = 16384

NUM_CORES = 2
NUM_SUBCORES = 16
NUM_WORKERS = NUM_CORES * NUM_SUBCORES
LANES = 16
CHUNK = 128
ROWS_PER_W = B // NUM_WORKERS
N_CHUNKS = ROWS_PER_W // CHUNK


def _calib_body(logits_hbm, topics_hbm, loga_hbm, b_hbm, out_hbm,
                idx_v, scale_v,
                rows0, rows1, rows2, rows3, logits0, logits1,
                sem_r0, sem_r1, sem_r2, sem_r3,
                sem_s0, sem_s1, sem_s2, sem_s3,
                sem_l0, sem_l1, sem_o0, sem_o1, sem_o2, sem_o3):
    wid = lax.axis_index("s") * NUM_CORES + lax.axis_index("c")
    base = wid * ROWS_PER_W

    rows = (rows0, rows1, rows2, rows3)
    logits_b = (logits0, logits1)
    sem_r = (sem_r0, sem_r1, sem_r2, sem_r3)
    sem_s = (sem_s0, sem_s1, sem_s2, sem_s3)
    sem_l = (sem_l0, sem_l1)
    sem_o = (sem_o0, sem_o1, sem_o2, sem_o3)

    def off(c):
        return pl.ds(pl.multiple_of(base + c * CHUNK, CHUNK), CHUNK)

    pltpu.sync_copy(topics_hbm.at[pl.ds(pl.multiple_of(base, CHUNK), ROWS_PER_W)], idx_v)

    rows_cp = [pltpu.async_copy(b_hbm.at[idx_v.at[pl.ds(c * CHUNK, CHUNK)]],
                                rows[c], sem_r[c])
               for c in range(N_CHUNKS)]
    scale_cp = [pltpu.async_copy(loga_hbm.at[idx_v.at[pl.ds(c * CHUNK, CHUNK)]],
                                 scale_v.at[pl.ds(c * CHUNK, CHUNK)], sem_s[c])
                for c in range(N_CHUNKS)]
    logits_cp = [None] * N_CHUNKS
    logits_cp[0] = pltpu.async_copy(logits_hbm.at[off(0)], logits_b[0], sem_l[0])
    logits_cp[1] = pltpu.async_copy(logits_hbm.at[off(1)], logits_b[1], sem_l[1])

    out_cp = [None] * N_CHUNKS
    for c in range(N_CHUNKS):
        p = c % 2
        scale_cp[c].wait()
        for j in range(CHUNK // LANES):
            sl = pl.ds(c * CHUNK + j * LANES, LANES)
            scale_v[sl] = jnp.exp(scale_v[sl])
        rows_cp[c].wait()
        logits_cp[c].wait()

        def group_body(g, _, c=c, p=p):
            sv = scale_v[pl.ds(c * CHUNK + g * LANES, LANES)]
            for rr in range(LANES):
                i = g * LANES + rr
                sc = sv[rr]
                for k in range(N_CLS // LANES):
                    sl = pl.ds(k * LANES, LANES)
                    plsc.addupdate(rows[c].at[i, sl], logits_b[p][i, sl] * sc)
            return 0

        lax.fori_loop(0, CHUNK // LANES, group_body, 0)
        out_cp[c] = pltpu.async_copy(rows[c], out_hbm.at[off(c)], sem_o[c])
        if c + 2 < N_CHUNKS:
            logits_cp[c + 2] = pltpu.async_copy(logits_hbm.at[off(c + 2)],
                                                logits_b[p], sem_l[p])

    for c in range(N_CHUNKS):
        out_cp[c].wait()


@jax.jit
def kernel(logits, topics, loga, b):
    topics = topics.astype(jnp.int32)
    run = pl.kernel(
        _calib_body,
        out_type=jax.ShapeDtypeStruct((B, N_CLS), jnp.float32),
        mesh=plsc.VectorSubcoreMesh(core_axis_name="c", subcore_axis_name="s"),
        scratch_types=[
            pltpu.VMEM((ROWS_PER_W,), jnp.int32),
            pltpu.VMEM((ROWS_PER_W,), jnp.float32),
        ] + [pltpu.VMEM((CHUNK, N_CLS), jnp.float32)] * 6
          + [pltpu.SemaphoreType.DMA] * 14,
    )
    return run(logits, topics, loga, b)
